# Initial kernel scaffold; baseline (speedup 1.0000x reference)
import jax
import jax.numpy as jnp
from jax.experimental import pallas as pl


def kernel(x, W):
    raise NotImplementedError("write your pallas kernel here")


# speedup vs baseline: 9.6175x; 9.6175x over previous
import functools

import jax
import jax.numpy as jnp
from jax.experimental import pallas as pl

_TOP_K = 32


def _xt_kernel(x_ref, w_ref, out_ref):
    xt = jnp.dot(x_ref[0], w_ref[...], preferred_element_type=jnp.float32)
    norm = jnp.sqrt(jnp.sum(xt * xt, axis=1, keepdims=True))
    out_ref[0] = xt / jnp.maximum(norm, 1e-12)


def _scores_kernel(xt_blk_ref, xt_all_ref, out_ref):
    xb = xt_blk_ref[0]
    xa = xt_all_ref[0]
    s = jax.lax.dot_general(
        xb, xa, (((1,), (1,)), ((), ())),
        preferred_element_type=jnp.float32)
    s = jnp.maximum(s, 0.0)

    bm = s.shape[0]
    lo = jnp.zeros((bm, 1), jnp.int32)
    hi = jnp.full((bm, 1), 0x40000000, jnp.int32)

    def body(_, lohi):
        lo, hi = lohi
        mid = (lo + hi) >> 1
        t = jax.lax.bitcast_convert_type(mid, jnp.float32)
        cnt = jnp.sum((s >= t).astype(jnp.int32), axis=1, keepdims=True)
        ge = cnt >= _TOP_K
        return jnp.where(ge, mid, lo), jnp.where(ge, hi, mid)

    lo, hi = jax.lax.fori_loop(0, 31, body, (lo, hi))
    thresh = jax.lax.bitcast_convert_type(lo, jnp.float32)

    sm = jnp.where(s >= thresh, s, 0.0)
    m = jnp.max(sm, axis=1, keepdims=True)
    e = jnp.exp(sm - m)
    out_ref[0] = e / jnp.sum(e, axis=1, keepdims=True)


@functools.partial(jax.jit, static_argnames=())
def kernel(x, W):
    B, N, D = x.shape
    bm1 = 256
    xt = pl.pallas_call(
        _xt_kernel,
        grid=(B, N // bm1),
        in_specs=[
            pl.BlockSpec((1, bm1, D), lambda b, i: (b, i, 0)),
            pl.BlockSpec((D, D), lambda b, i: (0, 0)),
        ],
        out_specs=pl.BlockSpec((1, bm1, D), lambda b, i: (b, i, 0)),
        out_shape=jax.ShapeDtypeStruct((B, N, D), jnp.float32),
    )(x, W)

    bm2 = 256
    out = pl.pallas_call(
        _scores_kernel,
        grid=(B, N // bm2),
        in_specs=[
            pl.BlockSpec((1, bm2, D), lambda b, i: (b, i, 0)),
            pl.BlockSpec((1, N, D), lambda b, i: (b, 0, 0)),
        ],
        out_specs=pl.BlockSpec((1, bm2, N), lambda b, i: (b, i, 0)),
        out_shape=jax.ShapeDtypeStruct((B, N, N), jnp.float32),
    )(xt, xt)
    return out
